# baseline (speedup 1.0000x reference)
import functools

import jax
import jax.numpy as jnp
from jax import lax
from jax.experimental import pallas as pl
from jax.experimental.pallas import tpu as pltpu
from jax.experimental.pallas import tpu_sc as plsc

D = 64
SCALE = 8.0
NC = 2
NS = 16
NW = NC * NS
L = 16
TR = 4


def _sc_embed(tokens, tableP):
    R, S = tokens.shape
    r_per_w = R // NW
    n_chunks = r_per_w // TR
    n_pairs = n_chunks // 2
    mesh = plsc.VectorSubcoreMesh(core_axis_name="c", subcore_axis_name="s")

    @functools.partial(
        pl.kernel,
        mesh=mesh,
        out_type=jax.ShapeDtypeStruct((R, S, 2 * D), jnp.float32),
        scratch_types=[
            pltpu.VMEM((r_per_w, S), jnp.int32),
            pltpu.VMEM((TR, S, 2 * D), jnp.float32),
            pltpu.VMEM((TR, S, 2 * D), jnp.float32),
            pltpu.SemaphoreType.DMA,
            pltpu.SemaphoreType.DMA,
            pltpu.SemaphoreType.DMA,
            pltpu.SemaphoreType.DMA,
        ],
        compiler_params=pltpu.CompilerParams(use_tc_tiling_on_sc=True),
    )
    def k(table_hbm, tok_hbm, out_hbm,
          idx_all, rows0, rows1, g0, g1, w0, w1):
        wid = lax.axis_index("s") * NC + lax.axis_index("c")
        base = wid * r_per_w
        rows = (rows0, rows1)
        gsem = (g0, g1)
        wsem = (w0, w1)

        pltpu.sync_copy(tok_hbm.at[pl.ds(base, r_per_w)], idx_all)

        def g_issue(c, b):
            for i in range(TR):
                pltpu.async_copy(
                    table_hbm.at[idx_all.at[c * TR + i]],
                    rows[b].at[i], gsem[b])

        def g_wait(c, b):
            for i in range(TR):
                pltpu.make_async_copy(
                    table_hbm.at[idx_all.at[c * TR + i]],
                    rows[b].at[i], gsem[b]).wait()

        def wb_issue(c, b):
            pltpu.async_copy(
                rows[b], out_hbm.at[pl.ds(base + c * TR, TR)], wsem[b])

        def wb_wait(c, b):
            pltpu.make_async_copy(
                rows[b], out_hbm.at[pl.ds(base + c * TR, TR)],
                wsem[b]).wait()

        def scale(b):
            r = rows[b]

            @plsc.parallel_loop(0, S, 1, unroll=2)
            def _sc(col):
                for i in range(TR):
                    for q in range(D // L):
                        r[i, col, pl.ds(q * L, L)] = (
                            r[i, col, pl.ds(q * L, L)] * SCALE)

        g_issue(0, 0)

        def body(j, carry):
            c0 = 2 * j

            @pl.when(j > 0)
            def _():
                wb_wait(c0 - 1, 1)

            g_issue(c0 + 1, 1)
            g_wait(c0, 0)
            scale(0)
            wb_issue(c0, 0)

            @pl.when(j < n_pairs - 1)
            def _():
                wb_wait(c0, 0)
                g_issue(c0 + 2, 0)

            g_wait(c0 + 1, 1)
            scale(1)
            wb_issue(c0 + 1, 1)
            return carry

        lax.fori_loop(0, n_pairs, body, 0)
        wb_wait(n_chunks - 2, 0)
        wb_wait(n_chunks - 1, 1)

    return k(tableP, tokens)


def kernel(tokens, table):
    tableP = jnp.concatenate([table, table], axis=1)
    out2 = _sc_embed(tokens, tableP)
    return out2[:, :, :D]


# speedup vs baseline: 1.0840x; 1.0840x over previous
import functools

import jax
import jax.numpy as jnp
from jax import lax
from jax.experimental import pallas as pl
from jax.experimental.pallas import tpu as pltpu
from jax.experimental.pallas import tpu_sc as plsc

D = 64
SCALE = 8.0
NC = 2
NS = 16
NW = NC * NS
L = 16
TR = 4


def _sc_embed(tokens, tableP):
    R, S = tokens.shape
    r_per_w = R // NW
    n_chunks = r_per_w // TR
    n_pairs = n_chunks // 2
    mesh = plsc.VectorSubcoreMesh(core_axis_name="c", subcore_axis_name="s")

    @functools.partial(
        pl.kernel,
        mesh=mesh,
        out_type=jax.ShapeDtypeStruct((R, S, 2 * D), jnp.float32),
        scratch_types=[
            pltpu.VMEM((r_per_w, S), jnp.int32),
            pltpu.VMEM((TR, S, 2 * D), jnp.float32),
            pltpu.VMEM((TR, S, 2 * D), jnp.float32),
            pltpu.SemaphoreType.DMA,
            pltpu.SemaphoreType.DMA,
            pltpu.SemaphoreType.DMA,
            pltpu.SemaphoreType.DMA,
        ],
        compiler_params=pltpu.CompilerParams(use_tc_tiling_on_sc=True),
    )
    def k(table_hbm, tok_hbm, out_hbm,
          idx_all, rows0, rows1, g0, g1, w0, w1):
        wid = lax.axis_index("s") * NC + lax.axis_index("c")
        base = wid * r_per_w
        rows = (rows0, rows1)
        gsem = (g0, g1)
        wsem = (w0, w1)

        pltpu.sync_copy(tok_hbm.at[pl.ds(base, r_per_w)], idx_all)

        def g_issue(c, b):
            for i in range(TR):
                pltpu.async_copy(
                    table_hbm.at[idx_all.at[c * TR + i]],
                    rows[b].at[i], gsem[b])

        def g_wait(c, b):
            for i in range(TR):
                pltpu.make_async_copy(
                    table_hbm.at[idx_all.at[c * TR + i]],
                    rows[b].at[i], gsem[b]).wait()

        def wb_issue(c, b):
            pltpu.async_copy(
                rows[b], out_hbm.at[pl.ds(base + c * TR, TR)], wsem[b])

        def wb_wait(c, b):
            pltpu.make_async_copy(
                rows[b], out_hbm.at[pl.ds(base + c * TR, TR)],
                wsem[b]).wait()

        def scale(b):
            r = rows[b]

            @plsc.parallel_loop(0, S, 1, unroll=2)
            def _sc(col):
                for i in range(TR):
                    for q in range(D // L):
                        r[i, col, pl.ds(q * L, L)] = (
                            r[i, col, pl.ds(q * L, L)] * SCALE)

        g_issue(0, 0)

        def body(j, carry):
            c0 = 2 * j

            @pl.when(j > 0)
            def _():
                wb_wait(c0 - 1, 1)

            g_issue(c0 + 1, 1)
            g_wait(c0, 0)
            scale(0)
            wb_issue(c0, 0)

            @pl.when(j < n_pairs - 1)
            def _():
                wb_wait(c0, 0)
                g_issue(c0 + 2, 0)

            g_wait(c0 + 1, 1)
            scale(1)
            wb_issue(c0 + 1, 1)
            return carry

        lax.fori_loop(0, n_pairs, body, 0)
        wb_wait(n_chunks - 2, 0)
        wb_wait(n_chunks - 1, 1)

    return k(tableP, tokens)


def kernel(tokens, table):
    V = table.shape[0]
    tableP = lax.dynamic_update_slice(
        jnp.zeros((V, 2 * D), jnp.float32), table, (0, 0))
    out2 = _sc_embed(tokens, tableP)
    return out2[:, :, :D]


# speedup vs baseline: 1.0886x; 1.0043x over previous
import functools

import jax
import jax.numpy as jnp
from jax import lax
from jax.experimental import pallas as pl
from jax.experimental.pallas import tpu as pltpu
from jax.experimental.pallas import tpu_sc as plsc

D = 64
SCALE = 8.0
NC = 2
NS = 16
NW = NC * NS
L = 16
TR = 2
NB = 4


def _sc_embed(tokens, tableP):
    R, S = tokens.shape
    r_per_w = R // NW
    n_chunks = r_per_w // TR
    n_iters = n_chunks // NB
    mesh = plsc.VectorSubcoreMesh(core_axis_name="c", subcore_axis_name="s")

    @functools.partial(
        pl.kernel,
        mesh=mesh,
        out_type=jax.ShapeDtypeStruct((R, S, 2 * D), jnp.float32),
        scratch_types=[
            pltpu.VMEM((r_per_w, S), jnp.int32),
            pltpu.VMEM((TR, S, 2 * D), jnp.float32),
            pltpu.VMEM((TR, S, 2 * D), jnp.float32),
            pltpu.VMEM((TR, S, 2 * D), jnp.float32),
            pltpu.VMEM((TR, S, 2 * D), jnp.float32),
            pltpu.SemaphoreType.DMA,
            pltpu.SemaphoreType.DMA,
            pltpu.SemaphoreType.DMA,
            pltpu.SemaphoreType.DMA,
            pltpu.SemaphoreType.DMA,
            pltpu.SemaphoreType.DMA,
            pltpu.SemaphoreType.DMA,
            pltpu.SemaphoreType.DMA,
        ],
        compiler_params=pltpu.CompilerParams(use_tc_tiling_on_sc=True),
    )
    def k(table_hbm, tok_hbm, out_hbm, idx_all,
          rows0, rows1, rows2, rows3,
          g0, g1, g2, g3, w0, w1, w2, w3):
        wid = lax.axis_index("s") * NC + lax.axis_index("c")
        base = wid * r_per_w
        rows = (rows0, rows1, rows2, rows3)
        gsem = (g0, g1, g2, g3)
        wsem = (w0, w1, w2, w3)

        pltpu.sync_copy(tok_hbm.at[pl.ds(base, r_per_w)], idx_all)

        def g_issue(c, b):
            for i in range(TR):
                pltpu.async_copy(
                    table_hbm.at[idx_all.at[c * TR + i]],
                    rows[b].at[i], gsem[b])

        def g_wait(c, b):
            for i in range(TR):
                pltpu.make_async_copy(
                    table_hbm.at[idx_all.at[c * TR + i]],
                    rows[b].at[i], gsem[b]).wait()

        def wb_issue(c, b):
            pltpu.async_copy(
                rows[b], out_hbm.at[pl.ds(base + c * TR, TR)], wsem[b])

        def wb_wait(c, b):
            pltpu.make_async_copy(
                rows[b], out_hbm.at[pl.ds(base + c * TR, TR)],
                wsem[b]).wait()

        def scale(b):
            r = rows[b]

            @plsc.parallel_loop(0, S, 1, unroll=2)
            def _sc(col):
                for i in range(TR):
                    for q in range(D // L):
                        r[i, col, pl.ds(q * L, L)] = (
                            r[i, col, pl.ds(q * L, L)] * SCALE)

        g_issue(0, 0)
        g_issue(1, 1)

        def body(j, carry):
            for t in range(NB):
                c = NB * j + t
                bn = (t + 2) % NB

                @pl.when(c + 2 < n_chunks)
                def _():
                    @pl.when(c >= 2)
                    def _():
                        wb_wait(c - 2, bn)

                    g_issue(c + 2, bn)

                g_wait(c, t)
                scale(t)
                wb_issue(c, t)
            return carry

        lax.fori_loop(0, n_iters, body, 0)
        for c in range(n_chunks - NB, n_chunks):
            wb_wait(c, c % NB)

    return k(tableP, tokens)


def kernel(tokens, table):
    tableP = jnp.pad(table, ((0, 0), (0, D)))
    out2 = _sc_embed(tokens, tableP)
    return out2[:, :, :D]
